# baseline (speedup 1.0000x reference)
import functools

import jax
import jax.numpy as jnp
from jax import lax
from jax.experimental import pallas as pl
from jax.experimental.pallas import tpu as pltpu
from jax.experimental.pallas import tpu_sc as plsc

_VOCAB = 100000
_D = 300
_DA = 256
_DB = _D - _DA
_B = 4096
_S = 200
_N = _B * _S
_NC = 2
_NS = 16
_NW = _NC * _NS
_PER_W = _N // _NW
_CH = 128
_N_CH = _PER_W // _CH


def _gather_body(idx_hbm, tab_hbm, tb_hbm, out3_hbm,
                 idx_v, rows_a, rows_b, tail_v, sem):
    out_hbm = out3_hbm.reshape(_N, _D)
    wid = lax.axis_index("s") * _NC + lax.axis_index("c")
    base = wid * _PER_W

    def chunk(i, carry):
        off = base + i * _CH
        pltpu.sync_copy(idx_hbm.at[pl.ds(off, _CH)], idx_v)
        ca = pltpu.async_copy(tab_hbm.at[idx_v, pl.ds(0, _DA)], rows_a, sem)
        cb = pltpu.async_copy(tb_hbm.at[idx_v], rows_b, sem)
        ca.wait()
        cb.wait()

        def row(r, c):
            tail_v[r, pl.ds(0, 16)] = rows_b[r, pl.ds(0, 16)]
            tail_v[r, pl.ds(16, 16)] = rows_b[r, pl.ds(16, 16)]
            tail_v[r, pl.ds(28, 16)] = rows_b[r, pl.ds(28, 16)]
            return c

        lax.fori_loop(0, _CH, row, 0)
        pltpu.sync_copy(rows_a, out_hbm.at[pl.ds(off, _CH), pl.ds(0, _DA)])
        pltpu.sync_copy(tail_v, out_hbm.at[pl.ds(off, _CH), pl.ds(_DA, _DB)])
        return carry

    lax.fori_loop(0, _N_CH, chunk, 0)


@functools.partial(jax.jit, static_argnums=())
def kernel(data, table):
    idx = data.reshape(_N).astype(jnp.int32)
    table_b = jnp.pad(table[:, _DA:], ((0, 0), (0, 128 - _DB)))
    mesh = plsc.VectorSubcoreMesh(
        core_axis_name="c", subcore_axis_name="s",
        num_cores=_NC, num_subcores=_NS)
    k = pl.kernel(
        _gather_body,
        out_type=jax.ShapeDtypeStruct((_B, _S, _D), jnp.float32),
        mesh=mesh,
        scratch_types=[
            pltpu.VMEM((_CH,), jnp.int32),
            pltpu.VMEM((_CH, _DA), jnp.float32),
            pltpu.VMEM((_CH, 128), jnp.float32),
            pltpu.VMEM((_CH, _DB), jnp.float32),
            pltpu.SemaphoreType.DMA,
        ],
    )
    return k(idx, table, table_b)


# speedup vs baseline: 1.1315x; 1.1315x over previous
import functools

import jax
import jax.numpy as jnp
from jax import lax
from jax.experimental import pallas as pl
from jax.experimental.pallas import tpu as pltpu
from jax.experimental.pallas import tpu_sc as plsc

_VOCAB = 100000
_D = 300
_DA = 256
_DB = _D - _DA
_B = 4096
_S = 200
_N = _B * _S
_NC = 2
_NS = 16
_NW = _NC * _NS
_PER_W = _N // _NW
_CH = 128
_N_CH = _PER_W // _CH
_N_PAIR = _N_CH // 2


def _gather_body(idx_hbm, tab_hbm, tb_hbm, out3_hbm,
                 idx_all, rows_a0, rows_a1, rows_b, tail_v,
                 sem_a0, sem_a1, sem_b):
    out_hbm = out3_hbm.reshape(_N, _D)
    wid = lax.axis_index("s") * _NC + lax.axis_index("c")
    base = wid * _PER_W

    pltpu.sync_copy(idx_hbm.at[pl.ds(base, _PER_W)], idx_all)

    def idx_sl(c):
        return idx_all.at[pl.ds(c * _CH, _CH)]

    def issue_a(c, rows_a, sem):
        return pltpu.async_copy(
            tab_hbm.at[idx_sl(c), pl.ds(0, _DA)], rows_a, sem)

    def fill_and_write2(c, rc):
        def row(r, carry):
            tail_v[r, pl.ds(0, 16)] = rc[r, pl.ds(0, 16)]
            tail_v[r, pl.ds(16, 16)] = rc[r, pl.ds(16, 16)]
            tail_v[r, pl.ds(28, 16)] = rc[r, pl.ds(28, 16)]
            return carry
        lax.fori_loop(0, _CH, row, 0)
        pltpu.sync_copy(tail_v,
                        out_hbm.at[pl.ds(base + c * _CH, _CH),
                                   pl.ds(_DA, _DB)])

    def write1(c, rows_a):
        pltpu.sync_copy(rows_a,
                        out_hbm.at[pl.ds(base + c * _CH, _CH),
                                   pl.ds(0, _DA)])

    issue_a(0, rows_a0, sem_a0)

    def pair(p, carry):
        c0 = 2 * p
        c1 = c0 + 1
        gb0 = pltpu.async_copy(tb_hbm.at[idx_sl(c0)], rows_b, sem_b)
        issue_a(c1, rows_a1, sem_a1)
        pltpu.make_async_copy(
            tab_hbm.at[idx_sl(c0), pl.ds(0, _DA)], rows_a0, sem_a0).wait()
        write1(c0, rows_a0)
        gb0.wait()
        fill_and_write2(c0, rows_b)
        gb1 = pltpu.async_copy(tb_hbm.at[idx_sl(c1)], rows_b, sem_b)

        @pl.when(p + 1 < _N_PAIR)
        def _():
            issue_a(c0 + 2, rows_a0, sem_a0)

        pltpu.make_async_copy(
            tab_hbm.at[idx_sl(c1), pl.ds(0, _DA)], rows_a1, sem_a1).wait()
        write1(c1, rows_a1)
        gb1.wait()
        fill_and_write2(c1, rows_b)
        return carry

    lax.fori_loop(0, _N_PAIR, pair, 0)


@functools.partial(jax.jit, static_argnums=())
def kernel(data, table):
    idx = data.reshape(_N).astype(jnp.int32)
    table_b = jnp.pad(table[:, _DA:], ((0, 0), (0, 128 - _DB)))
    mesh = plsc.VectorSubcoreMesh(
        core_axis_name="c", subcore_axis_name="s",
        num_cores=_NC, num_subcores=_NS)
    k = pl.kernel(
        _gather_body,
        out_type=jax.ShapeDtypeStruct((_B, _S, _D), jnp.float32),
        mesh=mesh,
        scratch_types=[
            pltpu.VMEM((_PER_W,), jnp.int32),
            pltpu.VMEM((_CH, _DA), jnp.float32),
            pltpu.VMEM((_CH, _DA), jnp.float32),
            pltpu.VMEM((_CH, 128), jnp.float32),
            pltpu.VMEM((_CH, _DB), jnp.float32),
            pltpu.SemaphoreType.DMA,
            pltpu.SemaphoreType.DMA,
            pltpu.SemaphoreType.DMA,
        ],
    )
    return k(idx, table, table_b)
